# Initial kernel scaffold; baseline (speedup 1.0000x reference)
import jax
import jax.numpy as jnp
from jax.experimental import pallas as pl


def kernel(x, edge_index, g0_W, g0_asrc, g0_adst, g0_b, g1_W, g1_asrc, g1_adst, g1_b, g2_W, g2_asrc, g2_adst, g2_b, lat_W1, lat_b1, lat_W2, lat_b2, dec_W, dec_b):
    raise NotImplementedError("write your pallas kernel here")


# speedup vs baseline: 1.0120x; 1.0120x over previous
import functools
import jax
import jax.numpy as jnp
from jax.experimental import pallas as pl
from jax.experimental.pallas import tpu as pltpu

N = 100000
E = 1600000
NODE_DIM = 16
EMB = 16
HEADS = 4
HID = 64
UNEMB = 16
OUT = 4
D = EMB * HEADS

ROWS = 2000
GRID = N // ROWS


def _expand_mats():
    return jnp.kron(jnp.eye(HEADS, dtype=jnp.float32), jnp.ones((1, EMB), jnp.float32))


def _head_mats(asrc, adst):
    m = jnp.kron(jnp.eye(HEADS, dtype=jnp.float32), jnp.ones((EMB, 1), jnp.float32))
    return asrc.reshape(D)[:, None] * m, adst.reshape(D)[:, None] * m


def _dense_in_body(msg_ref, den_ref, w_ref, as_ref, ad_ref, b_ref, sel_ref,
                   h_ref, asd_ref, mx_ref, *, first):
    if first:
        hin = msg_ref[...]
    else:
        den64 = jnp.dot(den_ref[...], sel_ref[...],
                        preferred_element_type=jnp.float32)
        hin = msg_ref[...] / (den64 + 1e-16) + b_ref[0:1, :]
    h = jnp.dot(hin, w_ref[...], preferred_element_type=jnp.float32)
    h_ref[...] = h
    a_s = jnp.dot(h, as_ref[...], preferred_element_type=jnp.float32)
    a_d = jnp.dot(h, ad_ref[...], preferred_element_type=jnp.float32)
    asd = jnp.concatenate([a_s, a_d], axis=1)
    asd_ref[...] = asd
    mx_ref[0, 0, :] = jnp.max(asd, axis=0)


def _dense_layer(msg, den, W, asrc, adst, b, first=False):
    A_s, A_d = _head_mats(asrc, adst)
    sel = _expand_mats()
    b2 = jnp.broadcast_to(b[None, :], (8, D))
    grid = GRID
    out = pl.pallas_call(
        functools.partial(_dense_in_body, first=first),
        grid=(grid,),
        in_specs=[
            pl.BlockSpec((ROWS, msg.shape[1]), lambda i: (i, 0)),
            pl.BlockSpec((ROWS, HEADS), lambda i: (i, 0)),
            pl.BlockSpec((msg.shape[1], D), lambda i: (0, 0)),
            pl.BlockSpec((D, HEADS), lambda i: (0, 0)),
            pl.BlockSpec((D, HEADS), lambda i: (0, 0)),
            pl.BlockSpec((8, D), lambda i: (0, 0)),
            pl.BlockSpec((HEADS, D), lambda i: (0, 0)),
        ],
        out_specs=[
            pl.BlockSpec((ROWS, D), lambda i: (i, 0)),
            pl.BlockSpec((ROWS, 2 * HEADS), lambda i: (i, 0)),
            pl.BlockSpec((1, 1, 2 * HEADS), lambda i: (i, 0, 0)),
        ],
        out_shape=[
            jax.ShapeDtypeStruct((N, D), jnp.float32),
            jax.ShapeDtypeStruct((N, 2 * HEADS), jnp.float32),
            jax.ShapeDtypeStruct((grid, 1, 2 * HEADS), jnp.float32),
        ],
    )(msg, den, W, A_s, A_d, b2, sel)
    h, asd, mx = out
    m = jnp.max(mx.reshape(grid, 2 * HEADS), axis=0)
    C = jnp.maximum(jnp.max(m[:HEADS]) + jnp.max(m[HEADS:]), 0.0)
    return h, asd, C


def _final_body(msg_ref, den_ref, b_ref, sel_ref, w1_ref, b1_ref, w2_ref,
                b2_ref, wd_ref, bd_ref, out_ref):
    den64 = jnp.dot(den_ref[...], sel_ref[...], preferred_element_type=jnp.float32)
    hin = msg_ref[...] / (den64 + 1e-16) + b_ref[0:1, :]
    h = jnp.maximum(jnp.dot(hin, w1_ref[...], preferred_element_type=jnp.float32)
                    + b1_ref[0:1, :], 0.0)
    h = jnp.dot(h, w2_ref[...], preferred_element_type=jnp.float32) + b2_ref[0:1, :]
    lg = jnp.dot(h, wd_ref[...], preferred_element_type=jnp.float32) + bd_ref[0:1, :]
    lg = lg - jnp.max(lg, axis=1, keepdims=True)
    ex = jnp.exp(lg)
    out_ref[...] = ex / jnp.sum(ex, axis=1, keepdims=True)


def _final_layer(msg, den, b, W1, b1, W2, b2, Wd, bd):
    sel = _expand_mats()
    bb = jnp.broadcast_to(b[None, :], (8, D))
    b1b = jnp.broadcast_to(b1[None, :], (8, HID))
    b2b = jnp.broadcast_to(b2[None, :], (8, UNEMB))
    bdb = jnp.broadcast_to(bd[None, :], (8, OUT))
    return pl.pallas_call(
        _final_body,
        grid=(GRID,),
        in_specs=[
            pl.BlockSpec((ROWS, D), lambda i: (i, 0)),
            pl.BlockSpec((ROWS, HEADS), lambda i: (i, 0)),
            pl.BlockSpec((8, D), lambda i: (0, 0)),
            pl.BlockSpec((HEADS, D), lambda i: (0, 0)),
            pl.BlockSpec((D, HID), lambda i: (0, 0)),
            pl.BlockSpec((8, HID), lambda i: (0, 0)),
            pl.BlockSpec((HID, UNEMB), lambda i: (0, 0)),
            pl.BlockSpec((8, UNEMB), lambda i: (0, 0)),
            pl.BlockSpec((UNEMB, OUT), lambda i: (0, 0)),
            pl.BlockSpec((8, OUT), lambda i: (0, 0)),
        ],
        out_specs=pl.BlockSpec((ROWS, OUT), lambda i: (i, 0)),
        out_shape=jax.ShapeDtypeStruct((N, OUT), jnp.float32),
    )(msg, den, bb, sel, W1, b1b, W2, b2b, Wd, bdb)


def _edge_phase(h, asd, C, src, dst):
    e = asd[src, :HEADS] + asd[dst, HEADS:]
    e = jnp.where(e > 0, e, 0.2 * e)
    ex = jnp.exp(e - C)
    den = jax.ops.segment_sum(ex, dst, num_segments=N)
    msg = jax.ops.segment_sum(
        (h[src].reshape(E, HEADS, EMB) * ex[:, :, None]).reshape(E, D),
        dst, num_segments=N)
    return msg, den


def kernel(x, edge_index, g0_W, g0_asrc, g0_adst, g0_b, g1_W, g1_asrc,
           g1_adst, g1_b, g2_W, g2_asrc, g2_adst, g2_b, lat_W1, lat_b1,
           lat_W2, lat_b2, dec_W, dec_b):
    src = edge_index[0]
    dst = edge_index[1]
    zden = jnp.zeros((N, HEADS), jnp.float32)

    h, asd, C = _dense_layer(x, zden, g0_W, g0_asrc, g0_adst, g0_b, first=True)
    msg, den = _edge_phase(h, asd, C, src, dst)
    h, asd, C = _dense_layer(msg, den, g1_W, g1_asrc, g1_adst, g0_b)
    msg, den = _edge_phase(h, asd, C, src, dst)
    h, asd, C = _dense_layer(msg, den, g2_W, g2_asrc, g2_adst, g1_b)
    msg, den = _edge_phase(h, asd, C, src, dst)
    return _final_layer(msg, den, g2_b, lat_W1, lat_b1, lat_W2, lat_b2,
                        dec_W, dec_b)


# speedup vs baseline: 29.2322x; 28.8856x over previous
import functools
import jax
import jax.numpy as jnp
from jax import lax
from jax.experimental import pallas as pl
from jax.experimental.pallas import tpu as pltpu
from jax.experimental.pallas import tpu_sc as plsc

N = 100000
E = 1600000
NODE_DIM = 16
EMB = 16
HEADS = 4
HID = 64
UNEMB = 16
OUT = 4
D = EMB * HEADS

ROWS = 2000
GRID = N // ROWS

CH = 80
EPT_A = E // 32
NCH_A = EPT_A // CH
EPT_B = E // 16
NCH_B = EPT_B // CH
BINR = 25000
BINP = 25088
TPR = BINP // 16

_MESH = plsc.VectorSubcoreMesh(core_axis_name="c", subcore_axis_name="s",
                               num_cores=2, num_subcores=16)


def _dg(x, idx):
    return x.at[idx].get(mode="promise_in_bounds")


def _pa_body(asd_hbm, src_hbm, dst_hbm, cvec_hbm, t_hbm, den_hbm,
             srcv, dstv, sstage, dstage, tflat, i0, i1, i2, i3,
             cbuf, zbuf, densh, sem1, sem2):
    cid = lax.axis_index("c")
    sid = lax.axis_index("s")
    gwid = cid * 16 + sid
    iota = lax.iota(jnp.int32, 16)
    zero16 = jnp.zeros((16,), jnp.float32)

    def zb(k, carry):
        zbuf[pl.ds(k * 16, 16)] = zero16
        return carry
    lax.fori_loop(0, 313, zb, 0)
    for j in range(5):
        pltpu.sync_copy(zbuf.at[pl.ds(0, 5000)],
                        densh.at[pl.ds(sid * 25000 + j * 5000, 5000)])
    pltpu.sync_copy(cvec_hbm, cbuf)
    plsc.subcore_barrier()
    cv = cbuf[...]

    shift = (iota & 3) + 4
    pat = iota & 3
    expi = [(iota >> 2) + 4 * m for m in range(4)]
    m0 = iota < 4
    m1 = iota < 8
    m2 = iota < 12
    irefs = [i0, i1, i2, i3]
    base = gwid * EPT_A

    def chunk(i, carry):
        gb = base + i * CH
        pltpu.sync_copy(src_hbm.at[pl.ds(gb, CH)], srcv)
        pltpu.sync_copy(dst_hbm.at[pl.ds(gb, CH)], dstv)
        c1 = pltpu.async_copy(asd_hbm.at[srcv], sstage, sem1)
        c2 = pltpu.async_copy(asd_hbm.at[dstv], dstage, sem2)
        c1.wait()
        c2.wait()
        for g in range(20):
            es = []
            for q in range(4):
                r = 4 * g + q
                es.append(sstage[r, :] + _dg(dstage[r, :], shift))
            ef = jnp.where(m0, _dg(es[0], pat),
                           jnp.where(m1, _dg(es[1], pat),
                                     jnp.where(m2, _dg(es[2], pat),
                                               _dg(es[3], pat))))
            ef = jnp.where(ef > 0, ef, 0.2 * ef)
            tflat[pl.ds(16 * g, 16)] = jnp.exp(ef - cv)
        for g in range(20):
            dvv = dstv[pl.ds(16 * (g // 4), 16)]
            idx = _dg(dvv, expi[g % 4]) * 4 + pat
            irefs[g // 5][pl.ds(16 * (g % 5), 16)] = idx
        for j in range(4):
            pltpu.sync_copy(tflat.at[pl.ds(CH * j, CH)],
                            densh.at[irefs[j]], add=True)
        pltpu.sync_copy(tflat.at[pl.ds(0, 4 * CH)],
                        t_hbm.at[pl.ds(gb * 4, 4 * CH)])
        return carry
    lax.fori_loop(0, NCH_A, chunk, 0)
    plsc.subcore_barrier()
    pltpu.sync_copy(densh.at[pl.ds(sid * 25000, 25000)],
                    den_hbm.at[pl.ds(cid * (4 * N) + sid * 25000, 25000)])


@functools.partial(
    pl.kernel,
    out_type=[jax.ShapeDtypeStruct((4 * E,), jnp.float32),
              jax.ShapeDtypeStruct((2 * 4 * N,), jnp.float32)],
    mesh=_MESH,
    scratch_types=[
        pltpu.VMEM((CH,), jnp.int32),
        pltpu.VMEM((CH,), jnp.int32),
        pltpu.VMEM((CH, 16), jnp.float32),
        pltpu.VMEM((CH, 16), jnp.float32),
        pltpu.VMEM((4 * CH,), jnp.float32),
        pltpu.VMEM((CH,), jnp.int32),
        pltpu.VMEM((CH,), jnp.int32),
        pltpu.VMEM((CH,), jnp.int32),
        pltpu.VMEM((CH,), jnp.int32),
        pltpu.VMEM((16,), jnp.float32),
        pltpu.VMEM((5008,), jnp.float32),
        pltpu.VMEM_SHARED((4 * N,), jnp.float32),
        pltpu.SemaphoreType.DMA,
        pltpu.SemaphoreType.DMA,
    ],
    compiler_params=pltpu.CompilerParams(use_tc_tiling_on_sc=False),
)
def _pass_a(*args):
    _pa_body(*args)


def _pb_body(h_hbm, src_hbm, dst_hbm, t_hbm, msg_hbm,
             srcv, dstv, lidx, inbf, hstage, vstage, tst, zbuf2, acc, sem1):
    cid = lax.axis_index("c")
    sid = lax.axis_index("s")
    iota = lax.iota(jnp.int32, 16)
    zero16 = jnp.zeros((16,), jnp.float32)

    for r in range(196):
        for hh in range(4):
            zbuf2[r, pl.ds(16 * hh, 16)] = zero16

    expi = [(iota >> 2) + 4 * m for m in range(4)]
    splat = [jnp.full((16,), c, jnp.int32) for c in range(16)]
    base_t = sid * EPT_B

    for b in range(2):
        nbase = cid * 50000 + b * BINR
        for j in range(8):
            pltpu.sync_copy(zbuf2.at[...],
                            acc.at[pl.ds(sid * TPR + j * 196, 196), :])
        plsc.subcore_barrier()

        def chunk(i, carry):
            gb = base_t + i * CH
            pltpu.sync_copy(src_hbm.at[pl.ds(gb, CH)], srcv)
            pltpu.sync_copy(dst_hbm.at[pl.ds(gb, CH)], dstv)
            pltpu.sync_copy(t_hbm.at[pl.ds(gb * 4, 4 * CH)], tst)
            c1 = pltpu.async_copy(h_hbm.at[srcv], hstage, sem1)
            for g in range(5):
                dvv = dstv[pl.ds(16 * g, 16)]
                loc = dvv - nbase
                inb = (loc >= 0) & (loc < BINR)
                lidx[pl.ds(16 * g, 16)] = jnp.where(inb, loc, 0)
                inbf[pl.ds(16 * g, 16)] = jnp.where(inb, 1.0, 0.0)
            c1.wait()
            for g in range(20):
                tq = tst[pl.ds(16 * g, 16)]
                inbv = inbf[pl.ds(16 * (g // 4), 16)]
                tz = tq * _dg(inbv, expi[g % 4])
                for q in range(4):
                    r = 4 * g + q
                    for hh in range(4):
                        sp = _dg(tz, splat[4 * q + hh])
                        vstage[r, pl.ds(16 * hh, 16)] = (
                            sp * hstage[r, pl.ds(16 * hh, 16)])
            pltpu.sync_copy(vstage.at[...], acc.at[lidx], add=True)
            return carry
        lax.fori_loop(0, NCH_B, chunk, 0)
        plsc.subcore_barrier()
        if True:
            rb = sid * TPR
            @pl.when(rb + TPR <= BINR)
            def _():
                pltpu.sync_copy(acc.at[pl.ds(rb, TPR), :],
                                msg_hbm.at[pl.ds(nbase + rb, TPR), :])
            @pl.when(rb + TPR > BINR)
            def _():
                pltpu.sync_copy(acc.at[pl.ds(rb, BINR - 15 * TPR), :],
                                msg_hbm.at[pl.ds(nbase + rb, BINR - 15 * TPR), :])
        plsc.subcore_barrier()


@functools.partial(
    pl.kernel,
    out_type=jax.ShapeDtypeStruct((N, D), jnp.float32),
    mesh=_MESH,
    scratch_types=[
        pltpu.VMEM((CH,), jnp.int32),
        pltpu.VMEM((CH,), jnp.int32),
        pltpu.VMEM((CH,), jnp.int32),
        pltpu.VMEM((CH,), jnp.float32),
        pltpu.VMEM((CH, D), jnp.float32),
        pltpu.VMEM((CH, D), jnp.float32),
        pltpu.VMEM((4 * CH,), jnp.float32),
        pltpu.VMEM((196, D), jnp.float32),
        pltpu.VMEM_SHARED((BINP, D), jnp.float32),
        pltpu.SemaphoreType.DMA,
    ],
    compiler_params=pltpu.CompilerParams(use_tc_tiling_on_sc=False),
)
def _pass_b(*args):
    _pb_body(*args)


def _expand_mat():
    return jnp.kron(jnp.eye(HEADS, dtype=jnp.float32),
                    jnp.ones((1, EMB), jnp.float32))


def _head_mats(asrc, adst):
    m = jnp.kron(jnp.eye(HEADS, dtype=jnp.float32),
                 jnp.ones((EMB, 1), jnp.float32))
    return asrc.reshape(D)[:, None] * m, adst.reshape(D)[:, None] * m


def _dense_in_body(msg_ref, d0_ref, d1_ref, w_ref, as_ref, ad_ref, b_ref,
                   sel_ref, h_ref, asd_ref, mx_ref, *, first):
    if first:
        hin = msg_ref[...]
    else:
        den4 = d0_ref[...] + d1_ref[...]
        den64 = jnp.dot(den4, sel_ref[...], preferred_element_type=jnp.float32)
        hin = msg_ref[...] / (den64 + 1e-16) + b_ref[0:1, :]
    h = jnp.dot(hin, w_ref[...], preferred_element_type=jnp.float32)
    h_ref[...] = h
    a_s = jnp.dot(h, as_ref[...], preferred_element_type=jnp.float32)
    a_d = jnp.dot(h, ad_ref[...], preferred_element_type=jnp.float32)
    asd = jnp.concatenate(
        [a_s, a_d, jnp.zeros((a_s.shape[0], 8), jnp.float32)], axis=1)
    asd_ref[...] = asd
    mx_ref[0, 0, :] = jnp.max(asd, axis=0)


def _dense_layer(msg, d0, d1, W, asrc, adst, b, first=False):
    A_s, A_d = _head_mats(asrc, adst)
    sel = _expand_mat()
    b2 = jnp.broadcast_to(b[None, :], (8, D))
    out = pl.pallas_call(
        functools.partial(_dense_in_body, first=first),
        grid=(GRID,),
        in_specs=[
            pl.BlockSpec((ROWS, msg.shape[1]), lambda i: (i, 0)),
            pl.BlockSpec((ROWS, HEADS), lambda i: (i, 0)),
            pl.BlockSpec((ROWS, HEADS), lambda i: (i, 0)),
            pl.BlockSpec((msg.shape[1], D), lambda i: (0, 0)),
            pl.BlockSpec((D, HEADS), lambda i: (0, 0)),
            pl.BlockSpec((D, HEADS), lambda i: (0, 0)),
            pl.BlockSpec((8, D), lambda i: (0, 0)),
            pl.BlockSpec((HEADS, D), lambda i: (0, 0)),
        ],
        out_specs=[
            pl.BlockSpec((ROWS, D), lambda i: (i, 0)),
            pl.BlockSpec((ROWS, 16), lambda i: (i, 0)),
            pl.BlockSpec((1, 1, 16), lambda i: (i, 0, 0)),
        ],
        out_shape=[
            jax.ShapeDtypeStruct((N, D), jnp.float32),
            jax.ShapeDtypeStruct((N, 16), jnp.float32),
            jax.ShapeDtypeStruct((GRID, 1, 16), jnp.float32),
        ],
    )(msg, d0, d1, W, A_s, A_d, b2, sel)
    h, asd, mx = out
    m = jnp.max(mx.reshape(GRID, 16), axis=0)
    mas = jnp.max(m[:HEADS])
    mad = jnp.max(m[HEADS:2 * HEADS])
    C = jnp.maximum(jnp.maximum(mas + mad, 2.0 * mad), 0.0)
    return h, asd, C


def _final_body(msg_ref, d0_ref, d1_ref, b_ref, sel_ref, w1_ref, b1_ref,
                w2_ref, b2_ref, wd_ref, bd_ref, out_ref):
    den4 = d0_ref[...] + d1_ref[...]
    den64 = jnp.dot(den4, sel_ref[...], preferred_element_type=jnp.float32)
    hin = msg_ref[...] / (den64 + 1e-16) + b_ref[0:1, :]
    h = jnp.maximum(jnp.dot(hin, w1_ref[...],
                            preferred_element_type=jnp.float32)
                    + b1_ref[0:1, :], 0.0)
    h = jnp.dot(h, w2_ref[...], preferred_element_type=jnp.float32) + b2_ref[0:1, :]
    lg = jnp.dot(h, wd_ref[...], preferred_element_type=jnp.float32) + bd_ref[0:1, :]
    lg = lg - jnp.max(lg, axis=1, keepdims=True)
    ex = jnp.exp(lg)
    out_ref[...] = ex / jnp.sum(ex, axis=1, keepdims=True)


def _final_layer(msg, d0, d1, b, W1, b1, W2, b2, Wd, bd):
    sel = _expand_mat()
    bb = jnp.broadcast_to(b[None, :], (8, D))
    b1b = jnp.broadcast_to(b1[None, :], (8, HID))
    b2b = jnp.broadcast_to(b2[None, :], (8, UNEMB))
    bdb = jnp.broadcast_to(bd[None, :], (8, OUT))
    return pl.pallas_call(
        _final_body,
        grid=(GRID,),
        in_specs=[
            pl.BlockSpec((ROWS, D), lambda i: (i, 0)),
            pl.BlockSpec((ROWS, HEADS), lambda i: (i, 0)),
            pl.BlockSpec((ROWS, HEADS), lambda i: (i, 0)),
            pl.BlockSpec((8, D), lambda i: (0, 0)),
            pl.BlockSpec((HEADS, D), lambda i: (0, 0)),
            pl.BlockSpec((D, HID), lambda i: (0, 0)),
            pl.BlockSpec((8, HID), lambda i: (0, 0)),
            pl.BlockSpec((HID, UNEMB), lambda i: (0, 0)),
            pl.BlockSpec((8, UNEMB), lambda i: (0, 0)),
            pl.BlockSpec((UNEMB, OUT), lambda i: (0, 0)),
            pl.BlockSpec((8, OUT), lambda i: (0, 0)),
        ],
        out_specs=pl.BlockSpec((ROWS, OUT), lambda i: (i, 0)),
        out_shape=jax.ShapeDtypeStruct((N, OUT), jnp.float32),
    )(msg, d0, d1, bb, sel, W1, b1b, W2, b2b, Wd, bdb)


def _edge_phase(h, asd, C, src, dst):
    cvec = jnp.broadcast_to(C, (16,)).astype(jnp.float32)
    t, den = _pass_a(asd, src, dst, cvec)
    msg = _pass_b(h, src, dst, t)
    d0 = den[:4 * N].reshape(N, HEADS)
    d1 = den[4 * N:].reshape(N, HEADS)
    return msg, d0, d1


def kernel(x, edge_index, g0_W, g0_asrc, g0_adst, g0_b, g1_W, g1_asrc,
           g1_adst, g1_b, g2_W, g2_asrc, g2_adst, g2_b, lat_W1, lat_b1,
           lat_W2, lat_b2, dec_W, dec_b):
    src = edge_index[0]
    dst = edge_index[1]
    zden = jnp.zeros((N, HEADS), jnp.float32)

    h, asd, C = _dense_layer(x, zden, zden, g0_W, g0_asrc, g0_adst, g0_b,
                             first=True)
    msg, d0, d1 = _edge_phase(h, asd, C, src, dst)
    h, asd, C = _dense_layer(msg, d0, d1, g1_W, g1_asrc, g1_adst, g0_b)
    msg, d0, d1 = _edge_phase(h, asd, C, src, dst)
    h, asd, C = _dense_layer(msg, d0, d1, g2_W, g2_asrc, g2_adst, g1_b)
    msg, d0, d1 = _edge_phase(h, asd, C, src, dst)
    return _final_layer(msg, d0, d1, g2_b, lat_W1, lat_b1, lat_W2, lat_b2,
                        dec_W, dec_b)


# speedup vs baseline: 31.4267x; 1.0751x over previous
import functools
import jax
import jax.numpy as jnp
from jax import lax
from jax.experimental import pallas as pl
from jax.experimental.pallas import tpu as pltpu
from jax.experimental.pallas import tpu_sc as plsc

N = 100000
E = 1600000
NODE_DIM = 16
EMB = 16
HEADS = 4
HID = 64
UNEMB = 16
OUT = 4
D = EMB * HEADS

ROWS = 2000
GRID = N // ROWS

CH = 80
EPT_A = E // 32
NCH_A = EPT_A // CH
EPT_B = E // 16
NCH_B = EPT_B // CH
BINR = 25000
BINP = 25088
TPR = BINP // 16

_MESH = plsc.VectorSubcoreMesh(core_axis_name="c", subcore_axis_name="s",
                               num_cores=2, num_subcores=16)


def _dg(x, idx):
    return x.at[idx].get(mode="promise_in_bounds")


SC_SUB = 5
SCW = SC_SUB * CH
NSC_A = EPT_A // SCW
NSC_B = EPT_B // SCW
DENR = 100096
DPT = DENR // 16


def _pa_body(asd_hbm, src_hbm, dst_hbm, cvec_hbm, t_hbm,
             srcv, dstv, sstage, dstage, tbig, cbuf, sems):
    cid = lax.axis_index("c")
    sid = lax.axis_index("s")
    gwid = cid * 16 + sid
    iota = lax.iota(jnp.int32, 16)

    pltpu.sync_copy(cvec_hbm, cbuf)
    cv = cbuf[...]

    shift = (iota & 3) + 4
    pat = iota & 3
    m0 = iota < 4
    m1 = iota < 8
    m2 = iota < 12
    rbase = gwid * (EPT_A // CH)

    def emit_in(sc, pr):
        row0 = rbase + sc * SC_SUB
        pltpu.sync_copy(src_hbm.at[pl.ds(row0, SC_SUB), :], srcv.at[pr])
        pltpu.sync_copy(dst_hbm.at[pl.ds(row0, SC_SUB), :], dstv.at[pr])
        for j in range(SC_SUB):
            pltpu.async_copy(asd_hbm.at[srcv.at[pr, j]],
                             sstage.at[pr, pl.ds(CH * j, CH), :], sems.at[pr])
            pltpu.async_copy(asd_hbm.at[dstv.at[pr, j]],
                             dstage.at[pr, pl.ds(CH * j, CH), :], sems.at[pr])

    def wait_in(pr):
        for j in range(SC_SUB):
            pltpu.make_async_copy(
                asd_hbm.at[srcv.at[pr, j]],
                sstage.at[pr, pl.ds(CH * j, CH), :], sems.at[pr]).wait()
            pltpu.make_async_copy(
                asd_hbm.at[dstv.at[pr, j]],
                dstage.at[pr, pl.ds(CH * j, CH), :], sems.at[pr]).wait()

    emit_in(0, 0)

    def super_chunk(k, carry):
        par = k & 1

        @pl.when(k + 1 < NSC_A)
        def _():
            emit_in(k + 1, 1 - par)
        wait_in(par)

        def sub(j, carry2):
            for g in range(20):
                es = []
                for q in range(4):
                    r = CH * j + 4 * g + q
                    es.append(sstage[par, r, :] + _dg(dstage[par, r, :], shift))
                ef = jnp.where(m0, _dg(es[0], pat),
                               jnp.where(m1, _dg(es[1], pat),
                                         jnp.where(m2, _dg(es[2], pat),
                                                   _dg(es[3], pat))))
                ef = jnp.where(ef > 0, ef, 0.2 * ef)
                t = jnp.exp(ef - cv)
                tbig[pl.ds(320 * j + 16 * g, 16)] = t
            return carry2
        lax.fori_loop(0, SC_SUB, sub, 0)
        gb = (rbase + k * SC_SUB) * CH
        pltpu.sync_copy(tbig.at[...], t_hbm.at[pl.ds(gb * 4, 4 * SCW)])
        return carry
    lax.fori_loop(0, NSC_A, super_chunk, 0)


@functools.partial(
    pl.kernel,
    out_type=jax.ShapeDtypeStruct((4 * E,), jnp.float32),
    mesh=_MESH,
    scratch_types=[
        pltpu.VMEM((2, SC_SUB, CH), jnp.int32),
        pltpu.VMEM((2, SC_SUB, CH), jnp.int32),
        pltpu.VMEM((2, SCW, 16), jnp.float32),
        pltpu.VMEM((2, SCW, 16), jnp.float32),
        pltpu.VMEM((4 * SCW,), jnp.float32),
        pltpu.VMEM((16,), jnp.float32),
        pltpu.SemaphoreType.DMA((2,)),
    ],
    compiler_params=pltpu.CompilerParams(use_tc_tiling_on_sc=False),
)
def _pass_a(*args):
    _pa_body(*args)


DH = 32
MW = 48


def _pb_body(h0_hbm, h1_hbm, src_hbm, dst_hbm, t_hbm, msg0_hbm, msg1_hbm,
             srcv, dstv, lidx, inbf, hstage, vstage, tst, zbuf2, acc, sems):
    cid = lax.axis_index("c")
    sid = lax.axis_index("s")
    iota = lax.iota(jnp.int32, 16)
    zero16 = jnp.zeros((16,), jnp.float32)

    for r in range(98):
        for hh in range(3):
            zbuf2[r, pl.ds(16 * hh, 16)] = zero16

    expi = [(iota >> 2) + 4 * m for m in range(4)]
    splat = [jnp.full((16,), c, jnp.int32) for c in range(16)]
    rowp = [(iota & 3) + 4 * q for q in range(4)]
    rbase = sid * (EPT_B // CH)

    for b in range(2):
        for chh in range(2):
            h_hbm = h0_hbm if chh == 0 else h1_hbm
            msg_hbm = msg0_hbm if chh == 0 else msg1_hbm
            nbase = cid * 50000 + b * BINR

            def emit_in(sc, pr):
                row0 = rbase + sc * SC_SUB
                pltpu.sync_copy(src_hbm.at[pl.ds(row0, SC_SUB), :],
                                srcv.at[pr])
                pltpu.sync_copy(dst_hbm.at[pl.ds(row0, SC_SUB), :],
                                dstv.at[pr])
                pltpu.sync_copy(t_hbm.at[pl.ds(row0 * CH * 4, 4 * SCW)],
                                tst.at[pr])
                for j in range(SC_SUB):
                    pltpu.async_copy(h_hbm.at[srcv.at[pr, j]],
                                     hstage.at[pr, pl.ds(CH * j, CH), :],
                                     sems.at[pr])

            def wait_in(pr):
                for j in range(SC_SUB):
                    pltpu.make_async_copy(
                        h_hbm.at[srcv.at[pr, j]],
                        hstage.at[pr, pl.ds(CH * j, CH), :],
                        sems.at[pr]).wait()

            for j in range(16):
                pltpu.sync_copy(zbuf2.at[...],
                                acc.at[pl.ds(sid * TPR + j * 98, 98), :])
            plsc.subcore_barrier()
            emit_in(0, 0)

            def super_chunk(k, carry):
                par = k & 1

                @pl.when(k + 1 < NSC_B)
                def _():
                    emit_in(k + 1, 1 - par)
                wait_in(par)

                def sub(j, carry2):
                    for g in range(5):
                        dvv = dstv[par, j, pl.ds(16 * g, 16)]
                        loc = dvv - nbase
                        inb = (loc >= 0) & (loc < BINR)
                        inbf[pl.ds(16 * g, 16)] = jnp.where(
                            inb, jnp.ones((16,), jnp.float32), 0.0)
                        lidx[j, pl.ds(16 * g, 16)] = jnp.where(inb, loc, 0)
                    for g in range(20):
                        tq = tst[par, pl.ds(320 * j + 16 * g, 16)]
                        inbv = inbf[pl.ds(16 * (g // 4), 16)]
                        tz = tq * _dg(inbv, expi[g % 4])
                        for q in range(4):
                            r = CH * j + 4 * g + q
                            for hh in range(2):
                                sp = _dg(tz, splat[4 * q + 2 * chh + hh])
                                vstage[4 * g + q, pl.ds(16 * hh, 16)] = (
                                    sp * hstage[par, r, pl.ds(16 * hh, 16)])
                            vstage[4 * g + q, pl.ds(32, 16)] = _dg(tz, rowp[q])
                    pltpu.sync_copy(vstage.at[...], acc.at[lidx.at[j]],
                                    add=True)
                    return carry2
                lax.fori_loop(0, SC_SUB, sub, 0)
                return carry
            lax.fori_loop(0, NSC_B, super_chunk, 0)
            plsc.subcore_barrier()
            rb = sid * TPR
            @pl.when(rb + TPR <= BINR)
            def _():
                pltpu.sync_copy(acc.at[pl.ds(rb, TPR), :],
                                msg_hbm.at[pl.ds(nbase + rb, TPR), :])
            @pl.when(rb + TPR > BINR)
            def _():
                pltpu.sync_copy(
                    acc.at[pl.ds(rb, BINR - 15 * TPR), :],
                    msg_hbm.at[pl.ds(nbase + rb, BINR - 15 * TPR), :])
            plsc.subcore_barrier()


@functools.partial(
    pl.kernel,
    out_type=[jax.ShapeDtypeStruct((N, MW), jnp.float32),
              jax.ShapeDtypeStruct((N, MW), jnp.float32)],
    mesh=_MESH,
    scratch_types=[
        pltpu.VMEM((2, SC_SUB, CH), jnp.int32),
        pltpu.VMEM((2, SC_SUB, CH), jnp.int32),
        pltpu.VMEM((SC_SUB, CH), jnp.int32),
        pltpu.VMEM((CH,), jnp.float32),
        pltpu.VMEM((2, SCW, DH), jnp.float32),
        pltpu.VMEM((CH, MW), jnp.float32),
        pltpu.VMEM((2, 4 * SCW), jnp.float32),
        pltpu.VMEM((98, MW), jnp.float32),
        pltpu.VMEM_SHARED((BINP, MW), jnp.float32),
        pltpu.SemaphoreType.DMA((2,)),
    ],
    compiler_params=pltpu.CompilerParams(use_tc_tiling_on_sc=False),
)
def _pass_b(*args):
    _pb_body(*args)


def _expand_mat():
    return jnp.kron(jnp.eye(HEADS, dtype=jnp.float32),
                    jnp.ones((1, EMB), jnp.float32))


def _head_mats(asrc, adst):
    m = jnp.kron(jnp.eye(HEADS, dtype=jnp.float32),
                 jnp.ones((EMB, 1), jnp.float32))
    return asrc.reshape(D)[:, None] * m, adst.reshape(D)[:, None] * m


def _dense_in_body(m0_ref, m1_ref, w_ref, as_ref, ad_ref,
                   b_ref, sel_ref, h0_ref, h1_ref, asd_ref, mx_ref, *, first):
    if first:
        hin = m0_ref[...]
    else:
        den4 = m0_ref[:, DH:DH + 4]
        den64 = jnp.dot(den4, sel_ref[...], preferred_element_type=jnp.float32)
        msg = jnp.concatenate([m0_ref[:, :DH], m1_ref[:, :DH]], axis=1)
        hin = msg / (den64 + 1e-16) + b_ref[0:1, :]
    h = jnp.dot(hin, w_ref[...], preferred_element_type=jnp.float32)
    h0_ref[...] = h[:, :DH]
    h1_ref[...] = h[:, DH:]
    a_s = jnp.dot(h, as_ref[...], preferred_element_type=jnp.float32)
    a_d = jnp.dot(h, ad_ref[...], preferred_element_type=jnp.float32)
    asd = jnp.concatenate(
        [a_s, a_d, jnp.zeros((a_s.shape[0], 8), jnp.float32)], axis=1)
    asd_ref[...] = asd
    mx_ref[0, 0, :] = jnp.max(asd, axis=0)


def _dense_layer(m0, m1, W, asrc, adst, b, first=False):
    A_s, A_d = _head_mats(asrc, adst)
    sel = _expand_mat()
    b2 = jnp.broadcast_to(b[None, :], (8, D))
    out = pl.pallas_call(
        functools.partial(_dense_in_body, first=first),
        grid=(GRID,),
        in_specs=[
            pl.BlockSpec((ROWS, m0.shape[1]), lambda i: (i, 0)),
            pl.BlockSpec((ROWS, m1.shape[1]), lambda i: (i, 0)),
            pl.BlockSpec((NODE_DIM if first else D, D), lambda i: (0, 0)),
            pl.BlockSpec((D, HEADS), lambda i: (0, 0)),
            pl.BlockSpec((D, HEADS), lambda i: (0, 0)),
            pl.BlockSpec((8, D), lambda i: (0, 0)),
            pl.BlockSpec((HEADS, D), lambda i: (0, 0)),
        ],
        out_specs=[
            pl.BlockSpec((ROWS, DH), lambda i: (i, 0)),
            pl.BlockSpec((ROWS, DH), lambda i: (i, 0)),
            pl.BlockSpec((ROWS, 16), lambda i: (i, 0)),
            pl.BlockSpec((1, 1, 16), lambda i: (i, 0, 0)),
        ],
        out_shape=[
            jax.ShapeDtypeStruct((N, DH), jnp.float32),
            jax.ShapeDtypeStruct((N, DH), jnp.float32),
            jax.ShapeDtypeStruct((N, 16), jnp.float32),
            jax.ShapeDtypeStruct((GRID, 1, 16), jnp.float32),
        ],
    )(m0, m1, W, A_s, A_d, b2, sel)
    h0, h1, asd, mx = out
    m = jnp.max(mx.reshape(GRID, 16), axis=0)
    mas = jnp.max(m[:HEADS])
    mad = jnp.max(m[HEADS:2 * HEADS])
    C = jnp.maximum(jnp.maximum(mas + mad, 2.0 * mad), 0.0)
    return h0, h1, asd, C


def _final_body(m0_ref, m1_ref, b_ref, sel_ref, w1_ref,
                b1_ref, w2_ref, b2_ref, wd_ref, bd_ref, out_ref):
    den4 = m0_ref[:, DH:DH + 4]
    den64 = jnp.dot(den4, sel_ref[...], preferred_element_type=jnp.float32)
    msg = jnp.concatenate([m0_ref[:, :DH], m1_ref[:, :DH]], axis=1)
    hin = msg / (den64 + 1e-16) + b_ref[0:1, :]
    h = jnp.maximum(jnp.dot(hin, w1_ref[...],
                            preferred_element_type=jnp.float32)
                    + b1_ref[0:1, :], 0.0)
    h = jnp.dot(h, w2_ref[...], preferred_element_type=jnp.float32) + b2_ref[0:1, :]
    lg = jnp.dot(h, wd_ref[...], preferred_element_type=jnp.float32) + bd_ref[0:1, :]
    lg = lg - jnp.max(lg, axis=1, keepdims=True)
    ex = jnp.exp(lg)
    out_ref[...] = ex / jnp.sum(ex, axis=1, keepdims=True)


def _final_layer(m0, m1, b, W1, b1, W2, b2, Wd, bd):
    sel = _expand_mat()
    bb = jnp.broadcast_to(b[None, :], (8, D))
    b1b = jnp.broadcast_to(b1[None, :], (8, HID))
    b2b = jnp.broadcast_to(b2[None, :], (8, UNEMB))
    bdb = jnp.broadcast_to(bd[None, :], (8, OUT))
    return pl.pallas_call(
        _final_body,
        grid=(GRID,),
        in_specs=[
            pl.BlockSpec((ROWS, MW), lambda i: (i, 0)),
            pl.BlockSpec((ROWS, MW), lambda i: (i, 0)),
            pl.BlockSpec((8, D), lambda i: (0, 0)),
            pl.BlockSpec((HEADS, D), lambda i: (0, 0)),
            pl.BlockSpec((D, HID), lambda i: (0, 0)),
            pl.BlockSpec((8, HID), lambda i: (0, 0)),
            pl.BlockSpec((HID, UNEMB), lambda i: (0, 0)),
            pl.BlockSpec((8, UNEMB), lambda i: (0, 0)),
            pl.BlockSpec((UNEMB, OUT), lambda i: (0, 0)),
            pl.BlockSpec((8, OUT), lambda i: (0, 0)),
        ],
        out_specs=pl.BlockSpec((ROWS, OUT), lambda i: (i, 0)),
        out_shape=jax.ShapeDtypeStruct((N, OUT), jnp.float32),
    )(m0, m1, bb, sel, W1, b1b, W2, b2b, Wd, bdb)


def _edge_phase(h0, h1, asd, C, src2, dst2):
    cvec = jnp.broadcast_to(C, (16,)).astype(jnp.float32)
    t = _pass_a(asd, src2, dst2, cvec)
    m0, m1 = _pass_b(h0, h1, src2, dst2, t)
    return m0, m1


def kernel(x, edge_index, g0_W, g0_asrc, g0_adst, g0_b, g1_W, g1_asrc,
           g1_adst, g1_b, g2_W, g2_asrc, g2_adst, g2_b, lat_W1, lat_b1,
           lat_W2, lat_b2, dec_W, dec_b):
    src2 = edge_index[0].reshape(E // CH, CH)
    dst2 = edge_index[1].reshape(E // CH, CH)
    zden = jnp.zeros((N, 16), jnp.float32)

    h0, h1, asd, C = _dense_layer(x, zden, g0_W, g0_asrc,
                                  g0_adst, g0_b, first=True)
    m0, m1 = _edge_phase(h0, h1, asd, C, src2, dst2)
    h0, h1, asd, C = _dense_layer(m0, m1, g1_W, g1_asrc, g1_adst, g0_b)
    m0, m1 = _edge_phase(h0, h1, asd, C, src2, dst2)
    h0, h1, asd, C = _dense_layer(m0, m1, g2_W, g2_asrc, g2_adst, g1_b)
    m0, m1 = _edge_phase(h0, h1, asd, C, src2, dst2)
    return _final_layer(m0, m1, g2_b, lat_W1, lat_b1, lat_W2,
                        lat_b2, dec_W, dec_b)


# speedup vs baseline: 31.4443x; 1.0006x over previous
import functools
import jax
import jax.numpy as jnp
from jax import lax
from jax.experimental import pallas as pl
from jax.experimental.pallas import tpu as pltpu
from jax.experimental.pallas import tpu_sc as plsc

N = 100000
E = 1600000
NODE_DIM = 16
EMB = 16
HEADS = 4
HID = 64
UNEMB = 16
OUT = 4
D = EMB * HEADS

ROWS = 2000
GRID = N // ROWS

CH = 80
EPT_A = E // 32
EPT_B = E // 16
BINR = 25000
BINP = 25088
TPR = BINP // 16

_MESH = plsc.VectorSubcoreMesh(core_axis_name="c", subcore_axis_name="s",
                               num_cores=2, num_subcores=16)


def _dg(x, idx):
    return x.at[idx].get(mode="promise_in_bounds")


SC_SUB = 5
SCW = SC_SUB * CH
NSC_A = EPT_A // SCW
NSC_B = EPT_B // SCW


def _pa_body(asd_hbm, src_hbm, dst_hbm, cvec_hbm, t_hbm,
             srcv, dstv, sstage, dstage, tbig, cbuf, sems):
    cid = lax.axis_index("c")
    sid = lax.axis_index("s")
    gwid = cid * 16 + sid
    iota = lax.iota(jnp.int32, 16)

    pltpu.sync_copy(cvec_hbm, cbuf)
    cv = cbuf[...]

    shift = (iota & 3) + 4
    pat = iota & 3
    m0 = iota < 4
    m1 = iota < 8
    m2 = iota < 12
    rbase = gwid * (EPT_A // CH)

    def emit_in(sc, pr):
        row0 = rbase + sc * SC_SUB
        pltpu.sync_copy(src_hbm.at[pl.ds(row0, SC_SUB), :], srcv.at[pr])
        pltpu.sync_copy(dst_hbm.at[pl.ds(row0, SC_SUB), :], dstv.at[pr])
        for j in range(SC_SUB):
            pltpu.async_copy(asd_hbm.at[srcv.at[pr, j]],
                             sstage.at[pr, pl.ds(CH * j, CH), :], sems.at[pr])
            pltpu.async_copy(asd_hbm.at[dstv.at[pr, j]],
                             dstage.at[pr, pl.ds(CH * j, CH), :], sems.at[pr])

    def wait_in(pr):
        for j in range(SC_SUB):
            pltpu.make_async_copy(
                asd_hbm.at[srcv.at[pr, j]],
                sstage.at[pr, pl.ds(CH * j, CH), :], sems.at[pr]).wait()
            pltpu.make_async_copy(
                asd_hbm.at[dstv.at[pr, j]],
                dstage.at[pr, pl.ds(CH * j, CH), :], sems.at[pr]).wait()

    emit_in(0, 0)

    def super_chunk(k, carry):
        par = k & 1

        @pl.when(k + 1 < NSC_A)
        def _():
            emit_in(k + 1, 1 - par)
        wait_in(par)

        def sub(j, carry2):
            for g in range(20):
                es = []
                for q in range(4):
                    r = CH * j + 4 * g + q
                    es.append(sstage[par, r, :] + _dg(dstage[par, r, :], shift))
                ef = jnp.where(m0, _dg(es[0], pat),
                               jnp.where(m1, _dg(es[1], pat),
                                         jnp.where(m2, _dg(es[2], pat),
                                                   _dg(es[3], pat))))
                ef = jnp.where(ef > 0, ef, 0.2 * ef)
                t = jnp.exp(ef - cv)
                tbig[pl.ds(320 * j + 16 * g, 16)] = t
            return carry2
        lax.fori_loop(0, SC_SUB, sub, 0)
        gb = (rbase + k * SC_SUB) * CH
        pltpu.sync_copy(tbig.at[...], t_hbm.at[pl.ds(gb * 4, 4 * SCW)])
        return carry
    lax.fori_loop(0, NSC_A, super_chunk, 0)


@functools.partial(
    pl.kernel,
    out_type=jax.ShapeDtypeStruct((4 * E,), jnp.float32),
    mesh=_MESH,
    scratch_types=[
        pltpu.VMEM((2, SC_SUB, CH), jnp.int32),
        pltpu.VMEM((2, SC_SUB, CH), jnp.int32),
        pltpu.VMEM((2, SCW, 16), jnp.float32),
        pltpu.VMEM((2, SCW, 16), jnp.float32),
        pltpu.VMEM((4 * SCW,), jnp.float32),
        pltpu.VMEM((16,), jnp.float32),
        pltpu.SemaphoreType.DMA((2,)),
    ],
    compiler_params=pltpu.CompilerParams(use_tc_tiling_on_sc=False),
)
def _pass_a(*args):
    _pa_body(*args)


DH = 32
MW = 48


def _pb_body(h0_hbm, h1_hbm, src_hbm, dst_hbm, t_hbm, msg0_hbm, msg1_hbm,
             srcv, dstv, lidx, inbf, hstage, vstage, tst, zbuf2, acc, sems):
    cid = lax.axis_index("c")
    sid = lax.axis_index("s")
    iota = lax.iota(jnp.int32, 16)
    zero16 = jnp.zeros((16,), jnp.float32)

    for r in range(98):
        for hh in range(3):
            zbuf2[r, pl.ds(16 * hh, 16)] = zero16

    expi = [(iota >> 2) + 4 * m for m in range(4)]
    splat = [jnp.full((16,), c, jnp.int32) for c in range(16)]
    rowp = [(iota & 3) + 4 * q for q in range(4)]
    rbase = sid * (EPT_B // CH)

    for b in range(2):
        for chh in range(2):
            h_hbm = h0_hbm if chh == 0 else h1_hbm
            msg_hbm = msg0_hbm if chh == 0 else msg1_hbm
            nbase = cid * 50000 + b * BINR

            def emit_in(sc, pr):
                row0 = rbase + sc * SC_SUB
                pltpu.sync_copy(src_hbm.at[pl.ds(row0, SC_SUB), :],
                                srcv.at[pr])
                pltpu.sync_copy(dst_hbm.at[pl.ds(row0, SC_SUB), :],
                                dstv.at[pr])
                pltpu.sync_copy(t_hbm.at[pl.ds(row0 * CH * 4, 4 * SCW)],
                                tst.at[pr])
                for j in range(SC_SUB):
                    pltpu.async_copy(h_hbm.at[srcv.at[pr, j]],
                                     hstage.at[pr, pl.ds(CH * j, CH), :],
                                     sems.at[pr])

            def wait_in(pr):
                for j in range(SC_SUB):
                    pltpu.make_async_copy(
                        h_hbm.at[srcv.at[pr, j]],
                        hstage.at[pr, pl.ds(CH * j, CH), :],
                        sems.at[pr]).wait()

            for j in range(16):
                pltpu.sync_copy(zbuf2.at[...],
                                acc.at[pl.ds(sid * TPR + j * 98, 98), :])
            plsc.subcore_barrier()
            emit_in(0, 0)

            def super_chunk(k, carry):
                par = k & 1

                @pl.when(k + 1 < NSC_B)
                def _():
                    emit_in(k + 1, 1 - par)
                wait_in(par)

                def sub(j, carry2):
                    for g in range(5):
                        dvv = dstv[par, j, pl.ds(16 * g, 16)]
                        loc = dvv - nbase
                        inb = (loc >= 0) & (loc < BINR)
                        inbf[pl.ds(16 * g, 16)] = jnp.where(
                            inb, jnp.ones((16,), jnp.float32), 0.0)
                        lidx[j, pl.ds(16 * g, 16)] = jnp.where(inb, loc, 0)
                    for g in range(20):
                        tq = tst[par, pl.ds(320 * j + 16 * g, 16)]
                        inbv = inbf[pl.ds(16 * (g // 4), 16)]
                        tz = tq * _dg(inbv, expi[g % 4])
                        for q in range(4):
                            r = CH * j + 4 * g + q
                            for hh in range(2):
                                sp = _dg(tz, splat[4 * q + 2 * chh + hh])
                                vstage[4 * g + q, pl.ds(16 * hh, 16)] = (
                                    sp * hstage[par, r, pl.ds(16 * hh, 16)])
                            vstage[4 * g + q, pl.ds(32, 16)] = _dg(tz, rowp[q])
                    pltpu.sync_copy(vstage.at[...], acc.at[lidx.at[j]],
                                    add=True)
                    return carry2
                lax.fori_loop(0, SC_SUB, sub, 0)
                return carry
            lax.fori_loop(0, NSC_B, super_chunk, 0)
            plsc.subcore_barrier()
            rb = sid * TPR
            @pl.when(rb + TPR <= BINR)
            def _():
                pltpu.sync_copy(acc.at[pl.ds(rb, TPR), :],
                                msg_hbm.at[pl.ds(nbase + rb, TPR), :])
            @pl.when(rb + TPR > BINR)
            def _():
                pltpu.sync_copy(
                    acc.at[pl.ds(rb, BINR - 15 * TPR), :],
                    msg_hbm.at[pl.ds(nbase + rb, BINR - 15 * TPR), :])
            plsc.subcore_barrier()


@functools.partial(
    pl.kernel,
    out_type=[jax.ShapeDtypeStruct((N, MW), jnp.float32),
              jax.ShapeDtypeStruct((N, MW), jnp.float32)],
    mesh=_MESH,
    scratch_types=[
        pltpu.VMEM((2, SC_SUB, CH), jnp.int32),
        pltpu.VMEM((2, SC_SUB, CH), jnp.int32),
        pltpu.VMEM((SC_SUB, CH), jnp.int32),
        pltpu.VMEM((CH,), jnp.float32),
        pltpu.VMEM((2, SCW, DH), jnp.float32),
        pltpu.VMEM((CH, MW), jnp.float32),
        pltpu.VMEM((2, 4 * SCW), jnp.float32),
        pltpu.VMEM((98, MW), jnp.float32),
        pltpu.VMEM_SHARED((BINP, MW), jnp.float32),
        pltpu.SemaphoreType.DMA((2,)),
    ],
    compiler_params=pltpu.CompilerParams(use_tc_tiling_on_sc=False),
)
def _pass_b(*args):
    _pb_body(*args)


def _expand_mat():
    return jnp.kron(jnp.eye(HEADS, dtype=jnp.float32),
                    jnp.ones((1, EMB), jnp.float32))


def _head_mats(asrc, adst):
    m = jnp.kron(jnp.eye(HEADS, dtype=jnp.float32),
                 jnp.ones((EMB, 1), jnp.float32))
    return asrc.reshape(D)[:, None] * m, adst.reshape(D)[:, None] * m


def _dense_in_body(m0_ref, m1_ref, w_ref, as_ref, ad_ref,
                   b_ref, sel_ref, h0_ref, h1_ref, asd_ref, mx_ref, *, first):
    if first:
        hin = m0_ref[...]
    else:
        den4 = m0_ref[:, DH:DH + 4]
        den64 = jnp.dot(den4, sel_ref[...], preferred_element_type=jnp.float32)
        msg = jnp.concatenate([m0_ref[:, :DH], m1_ref[:, :DH]], axis=1)
        hin = msg / (den64 + 1e-16) + b_ref[0:1, :]
    h = jnp.dot(hin, w_ref[...], preferred_element_type=jnp.float32)
    h0_ref[...] = h[:, :DH]
    h1_ref[...] = h[:, DH:]
    a_s = jnp.dot(h, as_ref[...], preferred_element_type=jnp.float32)
    a_d = jnp.dot(h, ad_ref[...], preferred_element_type=jnp.float32)
    asd = jnp.concatenate(
        [a_s, a_d, jnp.zeros((a_s.shape[0], 8), jnp.float32)], axis=1)
    asd_ref[...] = asd
    mx_ref[0, 0, :] = jnp.max(asd, axis=0)


def _dense_layer(m0, m1, W, asrc, adst, b, first=False):
    A_s, A_d = _head_mats(asrc, adst)
    sel = _expand_mat()
    b2 = jnp.broadcast_to(b[None, :], (8, D))
    out = pl.pallas_call(
        functools.partial(_dense_in_body, first=first),
        grid=(GRID,),
        in_specs=[
            pl.BlockSpec((ROWS, m0.shape[1]), lambda i: (i, 0)),
            pl.BlockSpec((ROWS, m1.shape[1]), lambda i: (i, 0)),
            pl.BlockSpec((NODE_DIM if first else D, D), lambda i: (0, 0)),
            pl.BlockSpec((D, HEADS), lambda i: (0, 0)),
            pl.BlockSpec((D, HEADS), lambda i: (0, 0)),
            pl.BlockSpec((8, D), lambda i: (0, 0)),
            pl.BlockSpec((HEADS, D), lambda i: (0, 0)),
        ],
        out_specs=[
            pl.BlockSpec((ROWS, DH), lambda i: (i, 0)),
            pl.BlockSpec((ROWS, DH), lambda i: (i, 0)),
            pl.BlockSpec((ROWS, 16), lambda i: (i, 0)),
            pl.BlockSpec((1, 1, 16), lambda i: (i, 0, 0)),
        ],
        out_shape=[
            jax.ShapeDtypeStruct((N, DH), jnp.float32),
            jax.ShapeDtypeStruct((N, DH), jnp.float32),
            jax.ShapeDtypeStruct((N, 16), jnp.float32),
            jax.ShapeDtypeStruct((GRID, 1, 16), jnp.float32),
        ],
    )(m0, m1, W, A_s, A_d, b2, sel)
    h0, h1, asd, mx = out
    m = jnp.max(mx.reshape(GRID, 16), axis=0)
    mas = jnp.max(m[:HEADS])
    mad = jnp.max(m[HEADS:2 * HEADS])
    C = jnp.maximum(jnp.maximum(mas + mad, 2.0 * mad), 0.0)
    return h0, h1, asd, C


def _final_body(m0_ref, m1_ref, b_ref, sel_ref, w1_ref,
                b1_ref, w2_ref, b2_ref, wd_ref, bd_ref, out_ref):
    den4 = m0_ref[:, DH:DH + 4]
    den64 = jnp.dot(den4, sel_ref[...], preferred_element_type=jnp.float32)
    msg = jnp.concatenate([m0_ref[:, :DH], m1_ref[:, :DH]], axis=1)
    hin = msg / (den64 + 1e-16) + b_ref[0:1, :]
    h = jnp.maximum(jnp.dot(hin, w1_ref[...],
                            preferred_element_type=jnp.float32)
                    + b1_ref[0:1, :], 0.0)
    h = jnp.dot(h, w2_ref[...], preferred_element_type=jnp.float32) + b2_ref[0:1, :]
    lg = jnp.dot(h, wd_ref[...], preferred_element_type=jnp.float32) + bd_ref[0:1, :]
    lg = lg - jnp.max(lg, axis=1, keepdims=True)
    ex = jnp.exp(lg)
    out_ref[...] = ex / jnp.sum(ex, axis=1, keepdims=True)


def _final_layer(m0, m1, b, W1, b1, W2, b2, Wd, bd):
    sel = _expand_mat()
    bb = jnp.broadcast_to(b[None, :], (8, D))
    b1b = jnp.broadcast_to(b1[None, :], (8, HID))
    b2b = jnp.broadcast_to(b2[None, :], (8, UNEMB))
    bdb = jnp.broadcast_to(bd[None, :], (8, OUT))
    return pl.pallas_call(
        _final_body,
        grid=(GRID,),
        in_specs=[
            pl.BlockSpec((ROWS, MW), lambda i: (i, 0)),
            pl.BlockSpec((ROWS, MW), lambda i: (i, 0)),
            pl.BlockSpec((8, D), lambda i: (0, 0)),
            pl.BlockSpec((HEADS, D), lambda i: (0, 0)),
            pl.BlockSpec((D, HID), lambda i: (0, 0)),
            pl.BlockSpec((8, HID), lambda i: (0, 0)),
            pl.BlockSpec((HID, UNEMB), lambda i: (0, 0)),
            pl.BlockSpec((8, UNEMB), lambda i: (0, 0)),
            pl.BlockSpec((UNEMB, OUT), lambda i: (0, 0)),
            pl.BlockSpec((8, OUT), lambda i: (0, 0)),
        ],
        out_specs=pl.BlockSpec((ROWS, OUT), lambda i: (i, 0)),
        out_shape=jax.ShapeDtypeStruct((N, OUT), jnp.float32),
    )(m0, m1, bb, sel, W1, b1b, W2, b2b, Wd, bdb)


def _edge_phase(h0, h1, asd, C, src2, dst2):
    cvec = jnp.broadcast_to(C, (16,)).astype(jnp.float32)
    t = _pass_a(asd, src2, dst2, cvec)
    m0, m1 = _pass_b(h0, h1, src2, dst2, t)
    return m0, m1


def kernel(x, edge_index, g0_W, g0_asrc, g0_adst, g0_b, g1_W, g1_asrc,
           g1_adst, g1_b, g2_W, g2_asrc, g2_adst, g2_b, lat_W1, lat_b1,
           lat_W2, lat_b2, dec_W, dec_b):
    src2 = edge_index[0].reshape(E // CH, CH)
    dst2 = edge_index[1].reshape(E // CH, CH)
    zden = jnp.zeros((N, 16), jnp.float32)

    h0, h1, asd, C = _dense_layer(x, zden, g0_W, g0_asrc,
                                  g0_adst, g0_b, first=True)
    m0, m1 = _edge_phase(h0, h1, asd, C, src2, dst2)
    h0, h1, asd, C = _dense_layer(m0, m1, g1_W, g1_asrc, g1_adst, g0_b)
    m0, m1 = _edge_phase(h0, h1, asd, C, src2, dst2)
    h0, h1, asd, C = _dense_layer(m0, m1, g2_W, g2_asrc, g2_adst, g1_b)
    m0, m1 = _edge_phase(h0, h1, asd, C, src2, dst2)
    return _final_layer(m0, m1, g2_b, lat_W1, lat_b1, lat_W2,
                        lat_b2, dec_W, dec_b)
